# Initial kernel scaffold; baseline (speedup 1.0000x reference)
import functools

import jax
import jax.numpy as jnp
from jax import lax
from jax.experimental import pallas as pl
from jax.experimental.pallas import tpu as pltpu
from jax.experimental.pallas import tpu_sc as plsc

N = 10000
E = 320000
D = 128

NC = 2
NS = 16
NW = NC * NS
E_PER_W = E // NW
CH = 80
N_IT = E_PER_W // CH
ROWS_PER_TILE = N // NS

_mesh = plsc.VectorSubcoreMesh(core_axis_name="c", subcore_axis_name="s")


@functools.partial(
    pl.kernel,
    out_type=(
        jax.ShapeDtypeStruct((E, D), jnp.float32),
        jax.ShapeDtypeStruct((E, D), jnp.float32),
    ),
    mesh=_mesh,
    scratch_types=[
        pltpu.VMEM((CH,), jnp.int32),
        pltpu.VMEM((CH,), jnp.int32),
        pltpu.VMEM((CH, D), jnp.float32),
        pltpu.VMEM((CH, D), jnp.float32),
        pltpu.SemaphoreType.DMA,
        pltpu.SemaphoreType.DMA,
    ],
)
def _sc_gather(a_hbm, b_hbm, dst_hbm, src_hbm, ga_hbm, gb_hbm,
               idx_d, idx_s, bufa, bufb, sema, semb):
    wid = lax.axis_index("c") * NS + lax.axis_index("s")
    base = wid * E_PER_W

    def body(j, carry):
        off = base + j * CH
        pltpu.sync_copy(dst_hbm.at[pl.ds(off, CH)], idx_d)
        pltpu.sync_copy(src_hbm.at[pl.ds(off, CH)], idx_s)
        ca = pltpu.async_copy(a_hbm.at[idx_d], bufa, sema)
        cb = pltpu.async_copy(b_hbm.at[idx_s], bufb, semb)
        ca.wait()
        cb.wait()
        pltpu.sync_copy(bufa, ga_hbm.at[pl.ds(off, CH)])
        pltpu.sync_copy(bufb, gb_hbm.at[pl.ds(off, CH)])
        return carry

    lax.fori_loop(0, N_IT, body, 0)


@functools.partial(
    pl.kernel,
    out_type=jax.ShapeDtypeStruct((NC, N, D), jnp.float32),
    mesh=_mesh,
    scratch_types=[
        pltpu.VMEM((CH,), jnp.int32),
        pltpu.VMEM((CH, D), jnp.float32),
        pltpu.VMEM_SHARED((N, D), jnp.float32),
    ],
)
def _sc_scatter(m_hbm, dst_hbm, zeros_hbm, s_hbm, idx_d, bufm, acc):
    cid = lax.axis_index("c")
    sid = lax.axis_index("s")
    base = (cid * NS + sid) * E_PER_W
    row0 = sid * ROWS_PER_TILE
    pltpu.sync_copy(zeros_hbm, acc.at[pl.ds(row0, ROWS_PER_TILE)])
    plsc.subcore_barrier()

    def body(j, carry):
        off = base + j * CH
        pltpu.sync_copy(dst_hbm.at[pl.ds(off, CH)], idx_d)
        pltpu.sync_copy(m_hbm.at[pl.ds(off, CH)], bufm)
        pltpu.sync_copy(bufm, acc.at[idx_d], add=True)
        return carry

    lax.fori_loop(0, N_IT, body, 0)
    plsc.subcore_barrier()
    pltpu.sync_copy(acc.at[pl.ds(row0, ROWS_PER_TILE)],
                    s_hbm.at[cid, pl.ds(row0, ROWS_PER_TILE)])


@functools.partial(
    pl.kernel,
    out_type=jax.ShapeDtypeStruct((NC, N, 16), jnp.float32),
    mesh=_mesh,
    scratch_types=[
        pltpu.VMEM((CH,), jnp.int32),
        pltpu.VMEM((CH, 16), jnp.float32),
        pltpu.VMEM_SHARED((N, 16), jnp.float32),
    ],
)
def _sc_counts(dst_hbm, ones_hbm, zeros_hbm, c_hbm, idx_d, bufo, acc):
    cid = lax.axis_index("c")
    sid = lax.axis_index("s")
    base = (cid * NS + sid) * E_PER_W
    row0 = sid * ROWS_PER_TILE
    pltpu.sync_copy(zeros_hbm, acc.at[pl.ds(row0, ROWS_PER_TILE)])
    pltpu.sync_copy(ones_hbm, bufo)
    plsc.subcore_barrier()

    def body(j, carry):
        off = base + j * CH
        pltpu.sync_copy(dst_hbm.at[pl.ds(off, CH)], idx_d)
        pltpu.sync_copy(bufo, acc.at[idx_d], add=True)
        return carry

    lax.fori_loop(0, N_IT, body, 0)
    plsc.subcore_barrier()
    pltpu.sync_copy(acc.at[pl.ds(row0, ROWS_PER_TILE)],
                    c_hbm.at[cid, pl.ds(row0, ROWS_PER_TILE)])


BN = 1000
BE = 1280

_full = lambda shape: pl.BlockSpec(shape, lambda i: (0,) * len(shape))
_rows = lambda b, w: pl.BlockSpec((b, w), lambda i: (i, 0))


def _tc_node_tables(h, wa, ba, wb):
    def body(h_ref, wa_ref, ba_ref, wb_ref, a_ref, b_ref):
        hv = h_ref[...]
        a_ref[...] = jnp.dot(hv, wa_ref[...], preferred_element_type=jnp.float32) + ba_ref[...]
        b_ref[...] = jnp.dot(hv, wb_ref[...], preferred_element_type=jnp.float32)
    return pl.pallas_call(
        body,
        grid=(N // BN,),
        in_specs=[_rows(BN, D), _full((D, D)), _full((1, D)), _full((D, D))],
        out_specs=[_rows(BN, D), _rows(BN, D)],
        out_shape=(jax.ShapeDtypeStruct((N, D), jnp.float32),
                   jax.ShapeDtypeStruct((N, D), jnp.float32)),
    )(h, wa, ba, wb)


def _tc_edge_mlp(ga, gb, w_col, v_row, w2, b2):
    def body(ga_ref, gb_ref, w_ref, v_ref, w2_ref, b2_ref, m_ref):
        m1 = jnp.maximum(ga_ref[...] + gb_ref[...] + w_ref[...] * v_ref[...], 0.0)
        m2 = jnp.dot(m1, w2_ref[...], preferred_element_type=jnp.float32) + b2_ref[...]
        m_ref[...] = jnp.maximum(m2, 0.0)
    return pl.pallas_call(
        body,
        grid=(E // BE,),
        in_specs=[_rows(BE, D), _rows(BE, D), _rows(BE, 1),
                  _full((1, D)), _full((D, D)), _full((1, D))],
        out_specs=_rows(BE, D),
        out_shape=jax.ShapeDtypeStruct((E, D), jnp.float32),
    )(ga, gb, w_col, v_row, w2, b2)


def _node_core(h_ref, s0, s1, c0, c1, wn1a, wn1b, bn1, wn2, bn2):
    cnt = jnp.maximum(c0[...][:, :1] + c1[...][:, :1], 1.0)
    aggr = (s0[...] + s1[...]) * (1.0 / cnt)
    hv = h_ref[...]
    u = jnp.dot(hv, wn1a[...], preferred_element_type=jnp.float32)
    u = u + jnp.dot(aggr, wn1b[...], preferred_element_type=jnp.float32) + bn1[...]
    u = jnp.maximum(u, 0.0)
    hn = jnp.dot(u, wn2[...], preferred_element_type=jnp.float32) + bn2[...]
    return jnp.maximum(hn, 0.0)


def _tc_node_update(h, s0, s1, c0, c1, wn1a, wn1b, bn1, wn2, bn2, wa, ba, wb):
    def body(h_ref, s0, s1, c0, c1, wn1a, wn1b, bn1, wn2, bn2,
             wa_ref, ba_ref, wb_ref, h_out, a_out, b_out):
        hn = _node_core(h_ref, s0, s1, c0, c1, wn1a, wn1b, bn1, wn2, bn2)
        h_out[...] = hn
        a_out[...] = jnp.dot(hn, wa_ref[...], preferred_element_type=jnp.float32) + ba_ref[...]
        b_out[...] = jnp.dot(hn, wb_ref[...], preferred_element_type=jnp.float32)
    return pl.pallas_call(
        body,
        grid=(N // BN,),
        in_specs=[_rows(BN, D), _rows(BN, D), _rows(BN, D),
                  _rows(BN, 16), _rows(BN, 16),
                  _full((D, D)), _full((D, D)), _full((1, D)),
                  _full((D, D)), _full((1, D)),
                  _full((D, D)), _full((1, D)), _full((D, D))],
        out_specs=[_rows(BN, D), _rows(BN, D), _rows(BN, D)],
        out_shape=(jax.ShapeDtypeStruct((N, D), jnp.float32),
                   jax.ShapeDtypeStruct((N, D), jnp.float32),
                   jax.ShapeDtypeStruct((N, D), jnp.float32)),
    )(h, s0, s1, c0, c1, wn1a, wn1b, bn1, wn2, bn2, wa, ba, wb)


def _tc_node_final(h, s0, s1, c0, c1, wn1a, wn1b, bn1, wn2, bn2, g, bta):
    def body(h_ref, s0, s1, c0, c1, wn1a, wn1b, bn1, wn2, bn2,
             g_ref, bta_ref, y_out):
        hn = _node_core(h_ref, s0, s1, c0, c1, wn1a, wn1b, bn1, wn2, bn2)
        mu = jnp.mean(hn, axis=1, keepdims=True)
        dlt = hn - mu
        var = jnp.mean(dlt * dlt, axis=1, keepdims=True)
        y_out[...] = dlt * lax.rsqrt(var + 1e-5) * g_ref[...] + bta_ref[...]
    return pl.pallas_call(
        body,
        grid=(N // BN,),
        in_specs=[_rows(BN, D), _rows(BN, D), _rows(BN, D),
                  _rows(BN, 16), _rows(BN, 16),
                  _full((D, D)), _full((D, D)), _full((1, D)),
                  _full((D, D)), _full((1, D)),
                  _full((1, D)), _full((1, D))],
        out_specs=_rows(BN, D),
        out_shape=jax.ShapeDtypeStruct((N, D), jnp.float32),
    )(h, s0, s1, c0, c1, wn1a, wn1b, bn1, wn2, bn2, g, bta)


def kernel(x, edge_index, edge_weight, params):
    src = edge_index[0].astype(jnp.int32)
    dst = edge_index[1].astype(jnp.int32)
    w_col = edge_weight.reshape(E, 1)

    zeros_d = jnp.zeros((ROWS_PER_TILE, D), jnp.float32)
    zeros_16 = jnp.zeros((ROWS_PER_TILE, 16), jnp.float32)
    ones_16 = jnp.ones((CH, 16), jnp.float32)

    cpart = _sc_counts(dst, ones_16, zeros_16)
    c0, c1 = cpart[0], cpart[1]

    layers = params["layers"]

    h = x
    we1 = layers[0]["We1"]
    a, b = _tc_node_tables(h, we1[:D], layers[0]["be1"].reshape(1, D), we1[D:2 * D])
    for i, p in enumerate(layers):
        ga, gb = _sc_gather(a, b, dst, src)
        m = _tc_edge_mlp(ga, gb, w_col, p["We1"][2 * D:2 * D + 1],
                         p["We2"], p["be2"].reshape(1, D))
        spart = _sc_scatter(m, dst, zeros_d)
        s0, s1 = spart[0], spart[1]
        wn1 = p["Wn1"]
        common = (h, s0, s1, c0, c1, wn1[:D], wn1[D:], p["bn1"].reshape(1, D),
                  p["Wn2"], p["bn2"].reshape(1, D))
        if i + 1 < len(layers):
            nxt = layers[i + 1]
            we1n = nxt["We1"]
            h, a, b = _tc_node_update(*common, we1n[:D],
                                      nxt["be1"].reshape(1, D), we1n[D:2 * D])
        else:
            h = _tc_node_final(*common, params["ln_scale"].reshape(1, D),
                               params["ln_bias"].reshape(1, D))
    return h


# speedup vs baseline: 2.3099x; 2.3099x over previous
import functools

import jax
import jax.numpy as jnp
from jax import lax
from jax.experimental import pallas as pl
from jax.experimental.pallas import tpu as pltpu
from jax.experimental.pallas import tpu_sc as plsc

N = 10000
E = 320000
D = 128

NC = 2
NS = 16
NW = NC * NS
E_PER_W = E // NW
CH = 80
N_IT = E_PER_W // CH
N_PAD = 10240
ROWS_PER_TILE = N_PAD // NS

_mesh = plsc.VectorSubcoreMesh(core_axis_name="c", subcore_axis_name="s")


@functools.partial(
    pl.kernel,
    out_type=(
        jax.ShapeDtypeStruct((E, D), jnp.float32),
        jax.ShapeDtypeStruct((E, D), jnp.float32),
    ),
    mesh=_mesh,
    scratch_types=[
        pltpu.VMEM((CH,), jnp.int32),
        pltpu.VMEM((CH,), jnp.int32),
        pltpu.VMEM((CH, D), jnp.float32),
        pltpu.VMEM((CH, D), jnp.float32),
        pltpu.SemaphoreType.DMA,
        pltpu.SemaphoreType.DMA,
    ],
)
def _sc_gather(a_hbm, b_hbm, dst_hbm, src_hbm, tok_hbm, ga_hbm, gb_hbm,
               idx_d, idx_s, bufa, bufb, sema, semb):
    del tok_hbm
    wid = lax.axis_index("c") * NS + lax.axis_index("s")
    base = wid * E_PER_W

    def body(j, carry):
        off = base + j * CH
        pltpu.sync_copy(dst_hbm.at[pl.ds(off, CH)], idx_d)
        pltpu.sync_copy(src_hbm.at[pl.ds(off, CH)], idx_s)
        ca = pltpu.async_copy(a_hbm.at[idx_d], bufa, sema)
        cb = pltpu.async_copy(b_hbm.at[idx_s], bufb, semb)
        ca.wait()
        cb.wait()
        pltpu.sync_copy(bufa, ga_hbm.at[pl.ds(off, CH)])
        pltpu.sync_copy(bufb, gb_hbm.at[pl.ds(off, CH)])
        return carry

    lax.fori_loop(0, N_IT, body, 0)


@functools.partial(
    pl.kernel,
    out_type=jax.ShapeDtypeStruct((NC, N_PAD, D), jnp.float32),
    mesh=_mesh,
    scratch_types=[
        pltpu.VMEM((CH,), jnp.int32),
        pltpu.VMEM((CH, D), jnp.float32),
        pltpu.VMEM_SHARED((N_PAD, D), jnp.float32),
    ],
)
def _sc_scatter(m_hbm, dst_hbm, zeros_hbm, s_hbm, idx_d, bufm, acc):
    cid = lax.axis_index("c")
    sid = lax.axis_index("s")
    base = (cid * NS + sid) * E_PER_W
    row0 = sid * ROWS_PER_TILE
    pltpu.sync_copy(zeros_hbm, acc.at[pl.ds(row0, ROWS_PER_TILE)])
    plsc.subcore_barrier()

    def body(j, carry):
        off = base + j * CH
        pltpu.sync_copy(dst_hbm.at[pl.ds(off, CH)], idx_d)
        pltpu.sync_copy(m_hbm.at[pl.ds(off, CH)], bufm)
        pltpu.sync_copy(bufm, acc.at[idx_d], add=True)
        return carry

    lax.fori_loop(0, N_IT, body, 0)
    plsc.subcore_barrier()
    pltpu.sync_copy(acc.at[pl.ds(row0, ROWS_PER_TILE)],
                    s_hbm.at[cid, pl.ds(row0, ROWS_PER_TILE)])


@functools.partial(
    pl.kernel,
    out_type=jax.ShapeDtypeStruct((NC, N_PAD, D), jnp.float32),
    mesh=_mesh,
    scratch_types=[
        pltpu.VMEM((CH,), jnp.int32),
        pltpu.VMEM((CH, D), jnp.float32),
        pltpu.VMEM_SHARED((N_PAD, D), jnp.float32),
    ],
)
def _sc_counts(dst_hbm, ones_hbm, zeros_hbm, c_hbm, idx_d, bufo, acc):
    cid = lax.axis_index("c")
    sid = lax.axis_index("s")
    base = (cid * NS + sid) * E_PER_W
    row0 = sid * ROWS_PER_TILE
    pltpu.sync_copy(zeros_hbm, acc.at[pl.ds(row0, ROWS_PER_TILE)])
    pltpu.sync_copy(ones_hbm, bufo)
    plsc.subcore_barrier()

    def body(j, carry):
        off = base + j * CH
        pltpu.sync_copy(dst_hbm.at[pl.ds(off, CH)], idx_d)
        pltpu.sync_copy(bufo, acc.at[idx_d], add=True)
        return carry

    lax.fori_loop(0, N_IT, body, 0)
    plsc.subcore_barrier()
    pltpu.sync_copy(acc.at[pl.ds(row0, ROWS_PER_TILE)],
                    c_hbm.at[cid, pl.ds(row0, ROWS_PER_TILE)])


BN = 1000
BE = 1280

_full = lambda shape: pl.BlockSpec(shape, lambda i: (0,) * len(shape))
_rows = lambda b, w: pl.BlockSpec((b, w), lambda i: (i, 0))


def _tc_node_tables(h, wa, ba, wb):
    def body(h_ref, wa_ref, ba_ref, wb_ref, a_ref, b_ref):
        hv = h_ref[...]
        a_ref[...] = jnp.dot(hv, wa_ref[...], preferred_element_type=jnp.float32) + ba_ref[...]
        b_ref[...] = jnp.dot(hv, wb_ref[...], preferred_element_type=jnp.float32)
    return pl.pallas_call(
        body,
        grid=(N // BN,),
        in_specs=[_rows(BN, D), _full((D, D)), _full((1, D)), _full((D, D))],
        out_specs=[_rows(BN, D), _rows(BN, D)],
        out_shape=(jax.ShapeDtypeStruct((N, D), jnp.float32),
                   jax.ShapeDtypeStruct((N, D), jnp.float32)),
    )(h, wa, ba, wb)


def _tc_edge_mlp(ga, gb, w_col, v_row, w2, b2):
    def body(ga_ref, gb_ref, w_ref, v_ref, w2_ref, b2_ref, m_ref):
        m1 = jnp.maximum(ga_ref[...] + gb_ref[...] + w_ref[...] * v_ref[...], 0.0)
        m2 = jnp.dot(m1, w2_ref[...], preferred_element_type=jnp.float32) + b2_ref[...]
        m_ref[...] = jnp.maximum(m2, 0.0)
    return pl.pallas_call(
        body,
        grid=(E // BE,),
        in_specs=[_rows(BE, D), _rows(BE, D), _rows(BE, 1),
                  _full((1, D)), _full((D, D)), _full((1, D))],
        out_specs=_rows(BE, D),
        out_shape=jax.ShapeDtypeStruct((E, D), jnp.float32),
    )(ga, gb, w_col, v_row, w2, b2)


def _node_core(h_ref, s0, s1, c0, c1, wn1a, wn1b, bn1, wn2, bn2):
    cnt = jnp.maximum(c0[...][:, :1] + c1[...][:, :1], 1.0)
    aggr = (s0[...] + s1[...]) * (1.0 / cnt)
    hv = h_ref[...]
    u = jnp.dot(hv, wn1a[...], preferred_element_type=jnp.float32)
    u = u + jnp.dot(aggr, wn1b[...], preferred_element_type=jnp.float32) + bn1[...]
    u = jnp.maximum(u, 0.0)
    hn = jnp.dot(u, wn2[...], preferred_element_type=jnp.float32) + bn2[...]
    return jnp.maximum(hn, 0.0)


def _tc_node_update(h, s0, s1, c0, c1, wn1a, wn1b, bn1, wn2, bn2, wa, ba, wb):
    def body(h_ref, s0, s1, c0, c1, wn1a, wn1b, bn1, wn2, bn2,
             wa_ref, ba_ref, wb_ref, h_out, a_out, b_out):
        hn = _node_core(h_ref, s0, s1, c0, c1, wn1a, wn1b, bn1, wn2, bn2)
        h_out[...] = hn
        a_out[...] = jnp.dot(hn, wa_ref[...], preferred_element_type=jnp.float32) + ba_ref[...]
        b_out[...] = jnp.dot(hn, wb_ref[...], preferred_element_type=jnp.float32)
    return pl.pallas_call(
        body,
        grid=(N // BN,),
        in_specs=[_rows(BN, D), _rows(BN, D), _rows(BN, D),
                  _rows(BN, D), _rows(BN, D),
                  _full((D, D)), _full((D, D)), _full((1, D)),
                  _full((D, D)), _full((1, D)),
                  _full((D, D)), _full((1, D)), _full((D, D))],
        out_specs=[_rows(BN, D), _rows(BN, D), _rows(BN, D)],
        out_shape=(jax.ShapeDtypeStruct((N, D), jnp.float32),
                   jax.ShapeDtypeStruct((N, D), jnp.float32),
                   jax.ShapeDtypeStruct((N, D), jnp.float32)),
    )(h, s0, s1, c0, c1, wn1a, wn1b, bn1, wn2, bn2, wa, ba, wb)


def _tc_node_final(h, s0, s1, c0, c1, wn1a, wn1b, bn1, wn2, bn2, g, bta):
    def body(h_ref, s0, s1, c0, c1, wn1a, wn1b, bn1, wn2, bn2,
             g_ref, bta_ref, y_out):
        hn = _node_core(h_ref, s0, s1, c0, c1, wn1a, wn1b, bn1, wn2, bn2)
        mu = jnp.mean(hn, axis=1, keepdims=True)
        dlt = hn - mu
        var = jnp.mean(dlt * dlt, axis=1, keepdims=True)
        y_out[...] = dlt * lax.rsqrt(var + 1e-5) * g_ref[...] + bta_ref[...]
    return pl.pallas_call(
        body,
        grid=(N // BN,),
        in_specs=[_rows(BN, D), _rows(BN, D), _rows(BN, D),
                  _rows(BN, D), _rows(BN, D),
                  _full((D, D)), _full((D, D)), _full((1, D)),
                  _full((D, D)), _full((1, D)),
                  _full((1, D)), _full((1, D))],
        out_specs=_rows(BN, D),
        out_shape=jax.ShapeDtypeStruct((N, D), jnp.float32),
    )(h, s0, s1, c0, c1, wn1a, wn1b, bn1, wn2, bn2, g, bta)


def kernel(x, edge_index, edge_weight, params):
    src = edge_index[0].astype(jnp.int32)
    dst = edge_index[1].astype(jnp.int32)
    w_col = edge_weight.reshape(E, 1)

    zeros_d = jnp.zeros((ROWS_PER_TILE, D), jnp.float32)
    ones_d = jnp.ones((CH, D), jnp.float32)

    cpart = _sc_counts(dst, ones_d, zeros_d)
    c0, c1 = cpart[0, :N], cpart[1, :N]
    tok = cpart[0, :8]

    layers = params["layers"]

    h = x
    we1 = layers[0]["We1"]
    a, b = _tc_node_tables(h, we1[:D], layers[0]["be1"].reshape(1, D), we1[D:2 * D])
    for i, p in enumerate(layers):
        ga, gb = _sc_gather(a, b, dst, src, tok)
        m = _tc_edge_mlp(ga, gb, w_col, p["We1"][2 * D:2 * D + 1],
                         p["We2"], p["be2"].reshape(1, D))
        spart = _sc_scatter(m, dst, zeros_d)
        s0, s1 = spart[0, :N], spart[1, :N]
        wn1 = p["Wn1"]
        common = (h, s0, s1, c0, c1, wn1[:D], wn1[D:], p["bn1"].reshape(1, D),
                  p["Wn2"], p["bn2"].reshape(1, D))
        if i + 1 < len(layers):
            nxt = layers[i + 1]
            we1n = nxt["We1"]
            h, a, b = _tc_node_update(*common, we1n[:D],
                                      nxt["be1"].reshape(1, D), we1n[D:2 * D])
        else:
            h = _tc_node_final(*common, params["ln_scale"].reshape(1, D),
                               params["ln_bias"].reshape(1, D))
    return h


# speedup vs baseline: 3.0019x; 1.2995x over previous
import functools

import jax
import jax.numpy as jnp
from jax import lax
from jax.experimental import pallas as pl
from jax.experimental.pallas import tpu as pltpu
from jax.experimental.pallas import tpu_sc as plsc

N = 10000
E = 320000
D = 128

NC = 2
NS = 16
NW = NC * NS
E_PER_W = E // NW
CH = 80
N_IT = E_PER_W // CH
N_PAD = 10240
ROWS_PER_TILE = N_PAD // NS

_mesh = plsc.VectorSubcoreMesh(core_axis_name="c", subcore_axis_name="s")


@functools.partial(
    pl.kernel,
    out_type=(
        jax.ShapeDtypeStruct((E, D), jnp.float32),
        jax.ShapeDtypeStruct((E, D), jnp.float32),
    ),
    mesh=_mesh,
    scratch_types=[
        pltpu.VMEM((CH,), jnp.int32),
        pltpu.VMEM((CH,), jnp.int32),
        pltpu.VMEM((CH,), jnp.int32),
        pltpu.VMEM((CH,), jnp.int32),
        pltpu.VMEM((CH, D), jnp.float32),
        pltpu.VMEM((CH, D), jnp.float32),
        pltpu.VMEM((CH, D), jnp.float32),
        pltpu.VMEM((CH, D), jnp.float32),
        pltpu.SemaphoreType.DMA,
        pltpu.SemaphoreType.DMA,
        pltpu.SemaphoreType.DMA,
        pltpu.SemaphoreType.DMA,
    ],
)
def _sc_gather(a_hbm, b_hbm, dst_hbm, src_hbm, tok_hbm, ga_hbm, gb_hbm,
               idx_d0, idx_s0, idx_d1, idx_s1, bufa0, bufb0, bufa1, bufb1,
               sema0, semb0, sema1, semb1):
    del tok_hbm
    wid = lax.axis_index("c") * NS + lax.axis_index("s")
    base = wid * E_PER_W
    slots = ((idx_d0, idx_s0, bufa0, bufb0, sema0, semb0),
             (idx_d1, idx_s1, bufa1, bufb1, sema1, semb1))

    def issue(c, sl):
        idx_d, idx_s, bufa, bufb, sema, semb = slots[sl]
        off = base + c * CH
        pltpu.sync_copy(dst_hbm.at[pl.ds(off, CH)], idx_d)
        pltpu.sync_copy(src_hbm.at[pl.ds(off, CH)], idx_s)
        pltpu.async_copy(a_hbm.at[idx_d], bufa, sema)
        pltpu.async_copy(b_hbm.at[idx_s], bufb, semb)

    def drain(c, sl):
        idx_d, idx_s, bufa, bufb, sema, semb = slots[sl]
        off = base + c * CH
        pltpu.make_async_copy(a_hbm.at[idx_d], bufa, sema).wait()
        pltpu.make_async_copy(b_hbm.at[idx_s], bufb, semb).wait()
        pltpu.sync_copy(bufa, ga_hbm.at[pl.ds(off, CH)])
        pltpu.sync_copy(bufb, gb_hbm.at[pl.ds(off, CH)])

    issue(0, 0)

    def body(it, carry):
        c0 = 2 * it
        issue(c0 + 1, 1)
        drain(c0, 0)
        issue(c0 + 2, 0)
        drain(c0 + 1, 1)
        return carry

    lax.fori_loop(0, (N_IT - 1) // 2, body, 0)
    drain(N_IT - 1, 0)


@functools.partial(
    pl.kernel,
    out_type=jax.ShapeDtypeStruct((NC, N_PAD, D), jnp.float32),
    mesh=_mesh,
    scratch_types=[
        pltpu.VMEM((CH,), jnp.int32),
        pltpu.VMEM((CH,), jnp.int32),
        pltpu.VMEM((CH, D), jnp.float32),
        pltpu.VMEM((CH, D), jnp.float32),
        pltpu.VMEM_SHARED((N_PAD, D), jnp.float32),
        pltpu.SemaphoreType.DMA,
        pltpu.SemaphoreType.DMA,
    ],
)
def _sc_scatter(m_hbm, dst_hbm, zeros_hbm, s_hbm, idx_d0, idx_d1,
                bufm0, bufm1, acc, semm0, semm1):
    cid = lax.axis_index("c")
    sid = lax.axis_index("s")
    base = (cid * NS + sid) * E_PER_W
    row0 = sid * ROWS_PER_TILE
    pltpu.sync_copy(zeros_hbm, acc.at[pl.ds(row0, ROWS_PER_TILE)])
    plsc.subcore_barrier()

    slots = ((idx_d0, bufm0, semm0), (idx_d1, bufm1, semm1))

    def issue(c, sl):
        idx_d, bufm, semm = slots[sl]
        off = base + c * CH
        pltpu.sync_copy(dst_hbm.at[pl.ds(off, CH)], idx_d)
        pltpu.async_copy(m_hbm.at[pl.ds(off, CH)], bufm, semm)

    def process(c, sl):
        idx_d, bufm, semm = slots[sl]
        off = base + c * CH
        pltpu.make_async_copy(m_hbm.at[pl.ds(off, CH)], bufm, semm).wait()
        pltpu.sync_copy(bufm, acc.at[idx_d], add=True)

    issue(0, 0)

    def body(it, carry):
        c0 = 2 * it
        issue(c0 + 1, 1)
        process(c0, 0)
        issue(c0 + 2, 0)
        process(c0 + 1, 1)
        return carry

    lax.fori_loop(0, (N_IT - 1) // 2, body, 0)
    process(N_IT - 1, 0)
    plsc.subcore_barrier()
    pltpu.sync_copy(acc.at[pl.ds(row0, ROWS_PER_TILE)],
                    s_hbm.at[cid, pl.ds(row0, ROWS_PER_TILE)])


@functools.partial(
    pl.kernel,
    out_type=jax.ShapeDtypeStruct((NC, N_PAD, D), jnp.float32),
    mesh=_mesh,
    scratch_types=[
        pltpu.VMEM((CH,), jnp.int32),
        pltpu.VMEM((CH, D), jnp.float32),
        pltpu.VMEM_SHARED((N_PAD, D), jnp.float32),
    ],
)
def _sc_counts(dst_hbm, ones_hbm, zeros_hbm, c_hbm, idx_d, bufo, acc):
    cid = lax.axis_index("c")
    sid = lax.axis_index("s")
    base = (cid * NS + sid) * E_PER_W
    row0 = sid * ROWS_PER_TILE
    pltpu.sync_copy(zeros_hbm, acc.at[pl.ds(row0, ROWS_PER_TILE)])
    pltpu.sync_copy(ones_hbm, bufo)
    plsc.subcore_barrier()

    def body(j, carry):
        off = base + j * CH
        pltpu.sync_copy(dst_hbm.at[pl.ds(off, CH)], idx_d)
        pltpu.sync_copy(bufo, acc.at[idx_d], add=True)
        return carry

    lax.fori_loop(0, N_IT, body, 0)
    plsc.subcore_barrier()
    pltpu.sync_copy(acc.at[pl.ds(row0, ROWS_PER_TILE)],
                    c_hbm.at[cid, pl.ds(row0, ROWS_PER_TILE)])


BN = 1000
BE = 1280

_full = lambda shape: pl.BlockSpec(shape, lambda i: (0,) * len(shape))
_rows = lambda b, w: pl.BlockSpec((b, w), lambda i: (i, 0))


def _tc_node_tables(h, wa, ba, wb):
    def body(h_ref, wa_ref, ba_ref, wb_ref, a_ref, b_ref):
        hv = h_ref[...]
        a_ref[...] = jnp.dot(hv, wa_ref[...], preferred_element_type=jnp.float32) + ba_ref[...]
        b_ref[...] = jnp.dot(hv, wb_ref[...], preferred_element_type=jnp.float32)
    return pl.pallas_call(
        body,
        grid=(N // BN,),
        in_specs=[_rows(BN, D), _full((D, D)), _full((1, D)), _full((D, D))],
        out_specs=[_rows(BN, D), _rows(BN, D)],
        out_shape=(jax.ShapeDtypeStruct((N, D), jnp.float32),
                   jax.ShapeDtypeStruct((N, D), jnp.float32)),
    )(h, wa, ba, wb)


def _tc_edge_mlp(ga, gb, w_col, v_row, w2, b2):
    def body(ga_ref, gb_ref, w_ref, v_ref, w2_ref, b2_ref, m_ref):
        m1 = jnp.maximum(ga_ref[...] + gb_ref[...] + w_ref[...] * v_ref[...], 0.0)
        m2 = jnp.dot(m1, w2_ref[...], preferred_element_type=jnp.float32) + b2_ref[...]
        m_ref[...] = jnp.maximum(m2, 0.0)
    return pl.pallas_call(
        body,
        grid=(E // BE,),
        in_specs=[_rows(BE, D), _rows(BE, D), _rows(BE, 1),
                  _full((1, D)), _full((D, D)), _full((1, D))],
        out_specs=_rows(BE, D),
        out_shape=jax.ShapeDtypeStruct((E, D), jnp.float32),
    )(ga, gb, w_col, v_row, w2, b2)


def _node_core(h_ref, s0, s1, c0, c1, wn1a, wn1b, bn1, wn2, bn2):
    cnt = jnp.maximum(c0[...][:, :1] + c1[...][:, :1], 1.0)
    aggr = (s0[...] + s1[...]) * (1.0 / cnt)
    hv = h_ref[...]
    u = jnp.dot(hv, wn1a[...], preferred_element_type=jnp.float32)
    u = u + jnp.dot(aggr, wn1b[...], preferred_element_type=jnp.float32) + bn1[...]
    u = jnp.maximum(u, 0.0)
    hn = jnp.dot(u, wn2[...], preferred_element_type=jnp.float32) + bn2[...]
    return jnp.maximum(hn, 0.0)


def _tc_node_update(h, s0, s1, c0, c1, wn1a, wn1b, bn1, wn2, bn2, wa, ba, wb):
    def body(h_ref, s0, s1, c0, c1, wn1a, wn1b, bn1, wn2, bn2,
             wa_ref, ba_ref, wb_ref, h_out, a_out, b_out):
        hn = _node_core(h_ref, s0, s1, c0, c1, wn1a, wn1b, bn1, wn2, bn2)
        h_out[...] = hn
        a_out[...] = jnp.dot(hn, wa_ref[...], preferred_element_type=jnp.float32) + ba_ref[...]
        b_out[...] = jnp.dot(hn, wb_ref[...], preferred_element_type=jnp.float32)
    return pl.pallas_call(
        body,
        grid=(N // BN,),
        in_specs=[_rows(BN, D), _rows(BN, D), _rows(BN, D),
                  _rows(BN, D), _rows(BN, D),
                  _full((D, D)), _full((D, D)), _full((1, D)),
                  _full((D, D)), _full((1, D)),
                  _full((D, D)), _full((1, D)), _full((D, D))],
        out_specs=[_rows(BN, D), _rows(BN, D), _rows(BN, D)],
        out_shape=(jax.ShapeDtypeStruct((N, D), jnp.float32),
                   jax.ShapeDtypeStruct((N, D), jnp.float32),
                   jax.ShapeDtypeStruct((N, D), jnp.float32)),
    )(h, s0, s1, c0, c1, wn1a, wn1b, bn1, wn2, bn2, wa, ba, wb)


def _tc_node_final(h, s0, s1, c0, c1, wn1a, wn1b, bn1, wn2, bn2, g, bta):
    def body(h_ref, s0, s1, c0, c1, wn1a, wn1b, bn1, wn2, bn2,
             g_ref, bta_ref, y_out):
        hn = _node_core(h_ref, s0, s1, c0, c1, wn1a, wn1b, bn1, wn2, bn2)
        mu = jnp.mean(hn, axis=1, keepdims=True)
        dlt = hn - mu
        var = jnp.mean(dlt * dlt, axis=1, keepdims=True)
        y_out[...] = dlt * lax.rsqrt(var + 1e-5) * g_ref[...] + bta_ref[...]
    return pl.pallas_call(
        body,
        grid=(N // BN,),
        in_specs=[_rows(BN, D), _rows(BN, D), _rows(BN, D),
                  _rows(BN, D), _rows(BN, D),
                  _full((D, D)), _full((D, D)), _full((1, D)),
                  _full((D, D)), _full((1, D)),
                  _full((1, D)), _full((1, D))],
        out_specs=_rows(BN, D),
        out_shape=jax.ShapeDtypeStruct((N, D), jnp.float32),
    )(h, s0, s1, c0, c1, wn1a, wn1b, bn1, wn2, bn2, g, bta)


def kernel(x, edge_index, edge_weight, params):
    src = edge_index[0].astype(jnp.int32)
    dst = edge_index[1].astype(jnp.int32)
    w_col = edge_weight.reshape(E, 1)

    zeros_d = jnp.zeros((ROWS_PER_TILE, D), jnp.float32)
    ones_d = jnp.ones((CH, D), jnp.float32)

    cpart = _sc_counts(dst, ones_d, zeros_d)
    c0, c1 = cpart[0, :N], cpart[1, :N]
    tok = cpart[0, :8]

    layers = params["layers"]

    h = x
    we1 = layers[0]["We1"]
    a, b = _tc_node_tables(h, we1[:D], layers[0]["be1"].reshape(1, D), we1[D:2 * D])
    for i, p in enumerate(layers):
        ga, gb = _sc_gather(a, b, dst, src, tok)
        m = _tc_edge_mlp(ga, gb, w_col, p["We1"][2 * D:2 * D + 1],
                         p["We2"], p["be2"].reshape(1, D))
        spart = _sc_scatter(m, dst, zeros_d)
        s0, s1 = spart[0, :N], spart[1, :N]
        wn1 = p["Wn1"]
        common = (h, s0, s1, c0, c1, wn1[:D], wn1[D:], p["bn1"].reshape(1, D),
                  p["Wn2"], p["bn2"].reshape(1, D))
        if i + 1 < len(layers):
            nxt = layers[i + 1]
            we1n = nxt["We1"]
            h, a, b = _tc_node_update(*common, we1n[:D],
                                      nxt["be1"].reshape(1, D), we1n[D:2 * D])
        else:
            h = _tc_node_final(*common, params["ln_scale"].reshape(1, D),
                               params["ln_bias"].reshape(1, D))
    return h


# speedup vs baseline: 3.1932x; 1.0638x over previous
import functools

import jax
import jax.numpy as jnp
from jax import lax
from jax.experimental import pallas as pl
from jax.experimental.pallas import tpu as pltpu
from jax.experimental.pallas import tpu_sc as plsc

N = 10000
E = 320000
D = 128

NC = 2
NS = 16
NW = NC * NS
E_PER_W = E // NW
CH = 80
N_IT = E_PER_W // CH
N_PAD = 10240
ROWS_PER_TILE = N_PAD // NS

_mesh = plsc.VectorSubcoreMesh(core_axis_name="c", subcore_axis_name="s")


@functools.partial(
    pl.kernel,
    out_type=jax.ShapeDtypeStruct((E, D), jnp.float32),
    mesh=_mesh,
    scratch_types=[
        pltpu.VMEM((CH,), jnp.int32),
        pltpu.VMEM((CH,), jnp.int32),
        pltpu.VMEM((CH,), jnp.int32),
        pltpu.VMEM((CH,), jnp.int32),
        pltpu.VMEM((CH, D), jnp.float32),
        pltpu.VMEM((CH, D), jnp.float32),
        pltpu.VMEM((CH, D), jnp.float32),
        pltpu.VMEM((CH, D), jnp.float32),
        pltpu.SemaphoreType.DMA,
        pltpu.SemaphoreType.DMA,
        pltpu.SemaphoreType.DMA,
        pltpu.SemaphoreType.DMA,
    ],
)
def _sc_gather(a_hbm, b_hbm, dst_hbm, src_hbm, tok_hbm, g_hbm,
               idx_d0, idx_s0, idx_d1, idx_s1, bufa0, bufb0, bufa1, bufb1,
               sema0, semb0, sema1, semb1):
    del tok_hbm
    wid = lax.axis_index("c") * NS + lax.axis_index("s")
    base = wid * E_PER_W
    slots = ((idx_d0, idx_s0, bufa0, bufb0, sema0, semb0),
             (idx_d1, idx_s1, bufa1, bufb1, sema1, semb1))

    def issue(c, sl):
        idx_d, idx_s, bufa, bufb, sema, semb = slots[sl]
        off = base + c * CH
        pltpu.sync_copy(dst_hbm.at[pl.ds(off, CH)], idx_d)
        pltpu.sync_copy(src_hbm.at[pl.ds(off, CH)], idx_s)
        pltpu.async_copy(a_hbm.at[idx_d], bufa, sema)
        pltpu.async_copy(b_hbm.at[idx_s], bufb, semb)

    def drain(c, sl):
        idx_d, idx_s, bufa, bufb, sema, semb = slots[sl]
        off = base + c * CH
        pltpu.make_async_copy(a_hbm.at[idx_d], bufa, sema).wait()
        pltpu.make_async_copy(b_hbm.at[idx_s], bufb, semb).wait()

        def add_row(r, carry):
            for cc in range(D // 16):
                bufa[r, pl.ds(cc * 16, 16)] = (bufa[r, pl.ds(cc * 16, 16)]
                                               + bufb[r, pl.ds(cc * 16, 16)])
            return carry

        lax.fori_loop(0, CH, add_row, 0)
        pltpu.sync_copy(bufa, g_hbm.at[pl.ds(off, CH)])

    issue(0, 0)

    def body(it, carry):
        c0 = 2 * it
        issue(c0 + 1, 1)
        drain(c0, 0)
        issue(c0 + 2, 0)
        drain(c0 + 1, 1)
        return carry

    lax.fori_loop(0, (N_IT - 1) // 2, body, 0)
    drain(N_IT - 1, 0)


@functools.partial(
    pl.kernel,
    out_type=jax.ShapeDtypeStruct((NC, N_PAD, D), jnp.float32),
    mesh=_mesh,
    scratch_types=[
        pltpu.VMEM((CH,), jnp.int32),
        pltpu.VMEM((CH,), jnp.int32),
        pltpu.VMEM((CH, D), jnp.float32),
        pltpu.VMEM((CH, D), jnp.float32),
        pltpu.VMEM_SHARED((N_PAD, D), jnp.float32),
        pltpu.SemaphoreType.DMA,
        pltpu.SemaphoreType.DMA,
    ],
)
def _sc_scatter(m_hbm, dst_hbm, zeros_hbm, s_hbm, idx_d0, idx_d1,
                bufm0, bufm1, acc, semm0, semm1):
    cid = lax.axis_index("c")
    sid = lax.axis_index("s")
    base = (cid * NS + sid) * E_PER_W
    row0 = sid * ROWS_PER_TILE
    pltpu.sync_copy(zeros_hbm, acc.at[pl.ds(row0, ROWS_PER_TILE)])
    plsc.subcore_barrier()

    slots = ((idx_d0, bufm0, semm0), (idx_d1, bufm1, semm1))

    def issue(c, sl):
        idx_d, bufm, semm = slots[sl]
        off = base + c * CH
        pltpu.sync_copy(dst_hbm.at[pl.ds(off, CH)], idx_d)
        pltpu.async_copy(m_hbm.at[pl.ds(off, CH)], bufm, semm)

    def process(c, sl):
        idx_d, bufm, semm = slots[sl]
        off = base + c * CH
        pltpu.make_async_copy(m_hbm.at[pl.ds(off, CH)], bufm, semm).wait()
        pltpu.sync_copy(bufm, acc.at[idx_d], add=True)

    issue(0, 0)

    def body(it, carry):
        c0 = 2 * it
        issue(c0 + 1, 1)
        process(c0, 0)
        issue(c0 + 2, 0)
        process(c0 + 1, 1)
        return carry

    lax.fori_loop(0, (N_IT - 1) // 2, body, 0)
    process(N_IT - 1, 0)
    plsc.subcore_barrier()
    pltpu.sync_copy(acc.at[pl.ds(row0, ROWS_PER_TILE)],
                    s_hbm.at[cid, pl.ds(row0, ROWS_PER_TILE)])


@functools.partial(
    pl.kernel,
    out_type=jax.ShapeDtypeStruct((NC, N_PAD, D), jnp.float32),
    mesh=_mesh,
    scratch_types=[
        pltpu.VMEM((CH,), jnp.int32),
        pltpu.VMEM((CH, D), jnp.float32),
        pltpu.VMEM_SHARED((N_PAD, D), jnp.float32),
    ],
)
def _sc_counts(dst_hbm, ones_hbm, zeros_hbm, c_hbm, idx_d, bufo, acc):
    cid = lax.axis_index("c")
    sid = lax.axis_index("s")
    base = (cid * NS + sid) * E_PER_W
    row0 = sid * ROWS_PER_TILE
    pltpu.sync_copy(zeros_hbm, acc.at[pl.ds(row0, ROWS_PER_TILE)])
    pltpu.sync_copy(ones_hbm, bufo)
    plsc.subcore_barrier()

    def body(j, carry):
        off = base + j * CH
        pltpu.sync_copy(dst_hbm.at[pl.ds(off, CH)], idx_d)
        pltpu.sync_copy(bufo, acc.at[idx_d], add=True)
        return carry

    lax.fori_loop(0, N_IT, body, 0)
    plsc.subcore_barrier()
    pltpu.sync_copy(acc.at[pl.ds(row0, ROWS_PER_TILE)],
                    c_hbm.at[cid, pl.ds(row0, ROWS_PER_TILE)])


BN = 1000
BE = 1280

_full = lambda shape: pl.BlockSpec(shape, lambda i: (0,) * len(shape))
_rows = lambda b, w: pl.BlockSpec((b, w), lambda i: (i, 0))


def _tc_node_tables(h, wa, ba, wb):
    def body(h_ref, wa_ref, ba_ref, wb_ref, a_ref, b_ref):
        hv = h_ref[...]
        a_ref[...] = jnp.dot(hv, wa_ref[...], preferred_element_type=jnp.float32) + ba_ref[...]
        b_ref[...] = jnp.dot(hv, wb_ref[...], preferred_element_type=jnp.float32)
    return pl.pallas_call(
        body,
        grid=(N // BN,),
        in_specs=[_rows(BN, D), _full((D, D)), _full((1, D)), _full((D, D))],
        out_specs=[_rows(BN, D), _rows(BN, D)],
        out_shape=(jax.ShapeDtypeStruct((N, D), jnp.float32),
                   jax.ShapeDtypeStruct((N, D), jnp.float32)),
    )(h, wa, ba, wb)


def _tc_edge_mlp(g, w_col, v_row, w2, b2):
    def body(g_ref, w_ref, v_ref, w2_ref, b2_ref, m_ref):
        m1 = jnp.maximum(g_ref[...] + w_ref[...] * v_ref[...], 0.0)
        m2 = jnp.dot(m1, w2_ref[...], preferred_element_type=jnp.float32) + b2_ref[...]
        m_ref[...] = jnp.maximum(m2, 0.0)
    return pl.pallas_call(
        body,
        grid=(E // BE,),
        in_specs=[_rows(BE, D), _rows(BE, 1),
                  _full((1, D)), _full((D, D)), _full((1, D))],
        out_specs=_rows(BE, D),
        out_shape=jax.ShapeDtypeStruct((E, D), jnp.float32),
    )(g, w_col, v_row, w2, b2)


def _node_core(h_ref, s0, s1, c0, c1, wn1a, wn1b, bn1, wn2, bn2):
    cnt = jnp.maximum(c0[...][:, :1] + c1[...][:, :1], 1.0)
    aggr = (s0[...] + s1[...]) * (1.0 / cnt)
    hv = h_ref[...]
    u = jnp.dot(hv, wn1a[...], preferred_element_type=jnp.float32)
    u = u + jnp.dot(aggr, wn1b[...], preferred_element_type=jnp.float32) + bn1[...]
    u = jnp.maximum(u, 0.0)
    hn = jnp.dot(u, wn2[...], preferred_element_type=jnp.float32) + bn2[...]
    return jnp.maximum(hn, 0.0)


def _tc_node_update(h, s0, s1, c0, c1, wn1a, wn1b, bn1, wn2, bn2, wa, ba, wb):
    def body(h_ref, s0, s1, c0, c1, wn1a, wn1b, bn1, wn2, bn2,
             wa_ref, ba_ref, wb_ref, h_out, a_out, b_out):
        hn = _node_core(h_ref, s0, s1, c0, c1, wn1a, wn1b, bn1, wn2, bn2)
        h_out[...] = hn
        a_out[...] = jnp.dot(hn, wa_ref[...], preferred_element_type=jnp.float32) + ba_ref[...]
        b_out[...] = jnp.dot(hn, wb_ref[...], preferred_element_type=jnp.float32)
    return pl.pallas_call(
        body,
        grid=(N // BN,),
        in_specs=[_rows(BN, D), _rows(BN, D), _rows(BN, D),
                  _rows(BN, D), _rows(BN, D),
                  _full((D, D)), _full((D, D)), _full((1, D)),
                  _full((D, D)), _full((1, D)),
                  _full((D, D)), _full((1, D)), _full((D, D))],
        out_specs=[_rows(BN, D), _rows(BN, D), _rows(BN, D)],
        out_shape=(jax.ShapeDtypeStruct((N, D), jnp.float32),
                   jax.ShapeDtypeStruct((N, D), jnp.float32),
                   jax.ShapeDtypeStruct((N, D), jnp.float32)),
    )(h, s0, s1, c0, c1, wn1a, wn1b, bn1, wn2, bn2, wa, ba, wb)


def _tc_node_final(h, s0, s1, c0, c1, wn1a, wn1b, bn1, wn2, bn2, g, bta):
    def body(h_ref, s0, s1, c0, c1, wn1a, wn1b, bn1, wn2, bn2,
             g_ref, bta_ref, y_out):
        hn = _node_core(h_ref, s0, s1, c0, c1, wn1a, wn1b, bn1, wn2, bn2)
        mu = jnp.mean(hn, axis=1, keepdims=True)
        dlt = hn - mu
        var = jnp.mean(dlt * dlt, axis=1, keepdims=True)
        y_out[...] = dlt * lax.rsqrt(var + 1e-5) * g_ref[...] + bta_ref[...]
    return pl.pallas_call(
        body,
        grid=(N // BN,),
        in_specs=[_rows(BN, D), _rows(BN, D), _rows(BN, D),
                  _rows(BN, D), _rows(BN, D),
                  _full((D, D)), _full((D, D)), _full((1, D)),
                  _full((D, D)), _full((1, D)),
                  _full((1, D)), _full((1, D))],
        out_specs=_rows(BN, D),
        out_shape=jax.ShapeDtypeStruct((N, D), jnp.float32),
    )(h, s0, s1, c0, c1, wn1a, wn1b, bn1, wn2, bn2, g, bta)


def kernel(x, edge_index, edge_weight, params):
    src = edge_index[0].astype(jnp.int32)
    dst = edge_index[1].astype(jnp.int32)
    w_col = edge_weight.reshape(E, 1)

    zeros_d = jnp.zeros((ROWS_PER_TILE, D), jnp.float32)
    ones_d = jnp.ones((CH, D), jnp.float32)

    cpart = _sc_counts(dst, ones_d, zeros_d)
    c0, c1 = cpart[0, :N], cpart[1, :N]
    tok = cpart[0, :8]

    layers = params["layers"]

    h = x
    we1 = layers[0]["We1"]
    a, b = _tc_node_tables(h, we1[:D], layers[0]["be1"].reshape(1, D), we1[D:2 * D])
    for i, p in enumerate(layers):
        g = _sc_gather(a, b, dst, src, tok)
        m = _tc_edge_mlp(g, w_col, p["We1"][2 * D:2 * D + 1],
                         p["We2"], p["be2"].reshape(1, D))
        spart = _sc_scatter(m, dst, zeros_d)
        s0, s1 = spart[0, :N], spart[1, :N]
        wn1 = p["Wn1"]
        common = (h, s0, s1, c0, c1, wn1[:D], wn1[D:], p["bn1"].reshape(1, D),
                  p["Wn2"], p["bn2"].reshape(1, D))
        if i + 1 < len(layers):
            nxt = layers[i + 1]
            we1n = nxt["We1"]
            h, a, b = _tc_node_update(*common, we1n[:D],
                                      nxt["be1"].reshape(1, D), we1n[D:2 * D])
        else:
            h = _tc_node_final(*common, params["ln_scale"].reshape(1, D),
                               params["ln_bias"].reshape(1, D))
    return h


# speedup vs baseline: 3.3134x; 1.0376x over previous
import functools

import jax
import jax.numpy as jnp
from jax import lax
from jax.experimental import pallas as pl
from jax.experimental.pallas import tpu as pltpu
from jax.experimental.pallas import tpu_sc as plsc

N = 10000
E = 320000
D = 128

NC = 2
NS = 16
NW = NC * NS
E_PER_W = E // NW
CH = 80
N_IT = E_PER_W // CH
N_PAD = 10240
ROWS_PER_TILE = N_PAD // NS

_mesh = plsc.VectorSubcoreMesh(core_axis_name="c", subcore_axis_name="s")


@functools.partial(
    pl.kernel,
    out_type=jax.ShapeDtypeStruct((E, D), jnp.float32),
    mesh=_mesh,
    scratch_types=[
        pltpu.VMEM((CH,), jnp.int32),
        pltpu.VMEM((CH,), jnp.int32),
        pltpu.VMEM((CH,), jnp.int32),
        pltpu.VMEM((CH,), jnp.int32),
        pltpu.VMEM((CH, D), jnp.float32),
        pltpu.VMEM((CH, D), jnp.float32),
        pltpu.VMEM((CH, D), jnp.float32),
        pltpu.VMEM((CH, D), jnp.float32),
        pltpu.SemaphoreType.DMA,
        pltpu.SemaphoreType.DMA,
        pltpu.SemaphoreType.DMA,
        pltpu.SemaphoreType.DMA,
    ],
)
def _sc_gather(a_hbm, b_hbm, dst_hbm, src_hbm, tok_hbm, g_hbm,
               idx_d0, idx_s0, idx_d1, idx_s1, bufa0, bufb0, bufa1, bufb1,
               sema0, semb0, sema1, semb1):
    del tok_hbm
    wid = lax.axis_index("c") * NS + lax.axis_index("s")
    base = wid * E_PER_W
    slots = ((idx_d0, idx_s0, bufa0, bufb0, sema0, semb0),
             (idx_d1, idx_s1, bufa1, bufb1, sema1, semb1))

    def issue(c, sl):
        idx_d, idx_s, bufa, bufb, sema, semb = slots[sl]
        off = base + c * CH
        pltpu.sync_copy(dst_hbm.at[pl.ds(off, CH)], idx_d)
        pltpu.sync_copy(src_hbm.at[pl.ds(off, CH)], idx_s)
        pltpu.async_copy(a_hbm.at[idx_d], bufa, sema)
        pltpu.async_copy(b_hbm.at[idx_s], bufb, semb)

    def drain(c, sl):
        idx_d, idx_s, bufa, bufb, sema, semb = slots[sl]
        off = base + c * CH
        pltpu.make_async_copy(a_hbm.at[idx_d], bufa, sema).wait()
        pltpu.make_async_copy(b_hbm.at[idx_s], bufb, semb).wait()

        def add_row(r, carry):
            for cc in range(D // 16):
                bufa[r, pl.ds(cc * 16, 16)] = (bufa[r, pl.ds(cc * 16, 16)]
                                               + bufb[r, pl.ds(cc * 16, 16)])
            return carry

        lax.fori_loop(0, CH, add_row, 0)
        pltpu.sync_copy(bufa, g_hbm.at[pl.ds(off, CH)])

    issue(0, 0)

    def body(it, carry):
        c0 = 2 * it
        issue(c0 + 1, 1)
        drain(c0, 0)
        issue(c0 + 2, 0)
        drain(c0 + 1, 1)
        return carry

    lax.fori_loop(0, (N_IT - 1) // 2, body, 0)
    drain(N_IT - 1, 0)


@functools.partial(
    pl.kernel,
    out_type=jax.ShapeDtypeStruct((NC, N_PAD, D), jnp.float32),
    mesh=_mesh,
    scratch_types=[
        pltpu.VMEM((CH,), jnp.int32),
        pltpu.VMEM((CH,), jnp.int32),
        pltpu.VMEM((CH, D), jnp.float32),
        pltpu.VMEM((CH, D), jnp.float32),
        pltpu.VMEM_SHARED((N_PAD, D), jnp.float32),
        pltpu.SemaphoreType.DMA,
        pltpu.SemaphoreType.DMA,
    ],
)
def _sc_scatter(m_hbm, dst_hbm, zeros_hbm, s_hbm, idx_d0, idx_d1,
                bufm0, bufm1, acc, semm0, semm1):
    cid = lax.axis_index("c")
    sid = lax.axis_index("s")
    base = (cid * NS + sid) * E_PER_W
    row0 = sid * ROWS_PER_TILE
    pltpu.sync_copy(zeros_hbm, acc.at[pl.ds(row0, ROWS_PER_TILE)])
    plsc.subcore_barrier()

    slots = ((idx_d0, bufm0, semm0), (idx_d1, bufm1, semm1))

    def issue(c, sl):
        idx_d, bufm, semm = slots[sl]
        off = base + c * CH
        pltpu.sync_copy(dst_hbm.at[pl.ds(off, CH)], idx_d)
        pltpu.async_copy(m_hbm.at[pl.ds(off, CH)], bufm, semm)

    def process(c, sl):
        idx_d, bufm, semm = slots[sl]
        off = base + c * CH
        pltpu.make_async_copy(m_hbm.at[pl.ds(off, CH)], bufm, semm).wait()
        pltpu.sync_copy(bufm, acc.at[idx_d], add=True)

    issue(0, 0)

    def body(it, carry):
        c0 = 2 * it
        issue(c0 + 1, 1)
        process(c0, 0)
        issue(c0 + 2, 0)
        process(c0 + 1, 1)
        return carry

    lax.fori_loop(0, (N_IT - 1) // 2, body, 0)
    process(N_IT - 1, 0)
    plsc.subcore_barrier()
    pltpu.sync_copy(acc.at[pl.ds(row0, ROWS_PER_TILE)],
                    s_hbm.at[cid, pl.ds(row0, ROWS_PER_TILE)])


@functools.partial(
    pl.kernel,
    out_type=jax.ShapeDtypeStruct((NC, N_PAD, D), jnp.float32),
    mesh=_mesh,
    scratch_types=[
        pltpu.VMEM((CH,), jnp.int32),
        pltpu.VMEM((CH, D), jnp.float32),
        pltpu.VMEM_SHARED((N_PAD, D), jnp.float32),
    ],
)
def _sc_counts(dst_hbm, ones_hbm, zeros_hbm, c_hbm, idx_d, bufo, acc):
    cid = lax.axis_index("c")
    sid = lax.axis_index("s")
    base = (cid * NS + sid) * E_PER_W
    row0 = sid * ROWS_PER_TILE
    pltpu.sync_copy(zeros_hbm, acc.at[pl.ds(row0, ROWS_PER_TILE)])
    pltpu.sync_copy(ones_hbm, bufo)
    plsc.subcore_barrier()

    def body(j, carry):
        off = base + j * CH
        pltpu.sync_copy(dst_hbm.at[pl.ds(off, CH)], idx_d)
        pltpu.sync_copy(bufo, acc.at[idx_d], add=True)
        return carry

    lax.fori_loop(0, N_IT, body, 0)
    plsc.subcore_barrier()
    pltpu.sync_copy(acc.at[pl.ds(row0, ROWS_PER_TILE)],
                    c_hbm.at[cid, pl.ds(row0, ROWS_PER_TILE)])


BN = 1000
BE = 1280

_full = lambda shape: pl.BlockSpec(shape, lambda i: (0,) * len(shape))
_rows = lambda b, w: pl.BlockSpec((b, w), lambda i: (i, 0))


def _tc_node_tables(h, wa, ba, wb):
    def body(h_ref, wa_ref, ba_ref, wb_ref, a_ref, b_ref):
        hv = h_ref[...]
        a_ref[...] = jnp.dot(hv, wa_ref[...], preferred_element_type=jnp.float32) + ba_ref[...]
        b_ref[...] = jnp.dot(hv, wb_ref[...], preferred_element_type=jnp.float32)
    return pl.pallas_call(
        body,
        grid=(N // BN,),
        in_specs=[_rows(BN, D), _full((D, D)), _full((1, D)), _full((D, D))],
        out_specs=[_rows(BN, D), _rows(BN, D)],
        out_shape=(jax.ShapeDtypeStruct((N, D), jnp.float32),
                   jax.ShapeDtypeStruct((N, D), jnp.float32)),
    )(h, wa, ba, wb)


def _tc_edge_mlp(g, w_row, v_row, w2, b2):
    def body(g_ref, w_ref, v_ref, w2_ref, b2_ref, m_ref):
        wv = lax.dot_general(w_ref[0], v_ref[...],
                             (((0,), (0,)), ((), ())),
                             preferred_element_type=jnp.float32)
        m1 = jnp.maximum(g_ref[...] + wv, 0.0).astype(jnp.bfloat16)
        m2 = jnp.dot(m1, w2_ref[...], preferred_element_type=jnp.float32) + b2_ref[...]
        m_ref[...] = jnp.maximum(m2, 0.0)
    return pl.pallas_call(
        body,
        grid=(E // BE,),
        in_specs=[_rows(BE, D), pl.BlockSpec((1, 1, BE), lambda i: (i, 0, 0)),
                  _full((1, D)), _full((D, D)), _full((1, D))],
        out_specs=_rows(BE, D),
        out_shape=jax.ShapeDtypeStruct((E, D), jnp.float32),
    )(g, w_row, v_row, w2, b2)


def _node_core(h_ref, s0, s1, c0, c1, wn1a, wn1b, bn1, wn2, bn2):
    cnt = jnp.maximum(c0[...][:, :1] + c1[...][:, :1], 1.0)
    aggr = (s0[...] + s1[...]) * (1.0 / cnt)
    hv = h_ref[...]
    u = jnp.dot(hv, wn1a[...], preferred_element_type=jnp.float32)
    u = u + jnp.dot(aggr, wn1b[...], preferred_element_type=jnp.float32) + bn1[...]
    u = jnp.maximum(u, 0.0)
    hn = jnp.dot(u, wn2[...], preferred_element_type=jnp.float32) + bn2[...]
    return jnp.maximum(hn, 0.0)


def _tc_node_update(h, s0, s1, c0, c1, wn1a, wn1b, bn1, wn2, bn2, wa, ba, wb):
    def body(h_ref, s0, s1, c0, c1, wn1a, wn1b, bn1, wn2, bn2,
             wa_ref, ba_ref, wb_ref, h_out, a_out, b_out):
        hn = _node_core(h_ref, s0, s1, c0, c1, wn1a, wn1b, bn1, wn2, bn2)
        h_out[...] = hn
        a_out[...] = jnp.dot(hn, wa_ref[...], preferred_element_type=jnp.float32) + ba_ref[...]
        b_out[...] = jnp.dot(hn, wb_ref[...], preferred_element_type=jnp.float32)
    return pl.pallas_call(
        body,
        grid=(N // BN,),
        in_specs=[_rows(BN, D), _rows(BN, D), _rows(BN, D),
                  _rows(BN, D), _rows(BN, D),
                  _full((D, D)), _full((D, D)), _full((1, D)),
                  _full((D, D)), _full((1, D)),
                  _full((D, D)), _full((1, D)), _full((D, D))],
        out_specs=[_rows(BN, D), _rows(BN, D), _rows(BN, D)],
        out_shape=(jax.ShapeDtypeStruct((N, D), jnp.float32),
                   jax.ShapeDtypeStruct((N, D), jnp.float32),
                   jax.ShapeDtypeStruct((N, D), jnp.float32)),
    )(h, s0, s1, c0, c1, wn1a, wn1b, bn1, wn2, bn2, wa, ba, wb)


def _tc_node_final(h, s0, s1, c0, c1, wn1a, wn1b, bn1, wn2, bn2, g, bta):
    def body(h_ref, s0, s1, c0, c1, wn1a, wn1b, bn1, wn2, bn2,
             g_ref, bta_ref, y_out):
        hn = _node_core(h_ref, s0, s1, c0, c1, wn1a, wn1b, bn1, wn2, bn2)
        mu = jnp.mean(hn, axis=1, keepdims=True)
        dlt = hn - mu
        var = jnp.mean(dlt * dlt, axis=1, keepdims=True)
        y_out[...] = dlt * lax.rsqrt(var + 1e-5) * g_ref[...] + bta_ref[...]
    return pl.pallas_call(
        body,
        grid=(N // BN,),
        in_specs=[_rows(BN, D), _rows(BN, D), _rows(BN, D),
                  _rows(BN, D), _rows(BN, D),
                  _full((D, D)), _full((D, D)), _full((1, D)),
                  _full((D, D)), _full((1, D)),
                  _full((1, D)), _full((1, D))],
        out_specs=_rows(BN, D),
        out_shape=jax.ShapeDtypeStruct((N, D), jnp.float32),
    )(h, s0, s1, c0, c1, wn1a, wn1b, bn1, wn2, bn2, g, bta)


def kernel(x, edge_index, edge_weight, params):
    src = edge_index[0].astype(jnp.int32)
    dst = edge_index[1].astype(jnp.int32)
    w_row = edge_weight.reshape(E // BE, 1, BE)

    zeros_d = jnp.zeros((ROWS_PER_TILE, D), jnp.float32)
    ones_d = jnp.ones((CH, D), jnp.float32)

    cpart = _sc_counts(dst, ones_d, zeros_d)
    c0, c1 = cpart[0, :N], cpart[1, :N]
    tok = cpart[0, :8]

    layers = params["layers"]

    h = x
    we1 = layers[0]["We1"]
    a, b = _tc_node_tables(h, we1[:D], layers[0]["be1"].reshape(1, D), we1[D:2 * D])
    for i, p in enumerate(layers):
        g = _sc_gather(a, b, dst, src, tok)
        m = _tc_edge_mlp(g, w_row, p["We1"][2 * D:2 * D + 1],
                         p["We2"].astype(jnp.bfloat16), p["be2"].reshape(1, D))
        spart = _sc_scatter(m, dst, zeros_d)
        s0, s1 = spart[0, :N], spart[1, :N]
        wn1 = p["Wn1"]
        common = (h, s0, s1, c0, c1, wn1[:D], wn1[D:], p["bn1"].reshape(1, D),
                  p["Wn2"], p["bn2"].reshape(1, D))
        if i + 1 < len(layers):
            nxt = layers[i + 1]
            we1n = nxt["We1"]
            h, a, b = _tc_node_update(*common, we1n[:D],
                                      nxt["be1"].reshape(1, D), we1n[D:2 * D])
        else:
            h = _tc_node_final(*common, params["ln_scale"].reshape(1, D),
                               params["ln_bias"].reshape(1, D))
    return h
